# baseline (speedup 1.0000x reference)
import functools

import jax
import jax.numpy as jnp
from jax import lax
from jax.experimental import pallas as pl
from jax.experimental.pallas import tpu as pltpu
from jax.experimental.pallas import tpu_sc as plsc

N = 524288
D = 64
TAB = 64
BC = 32768

NC = 2
NS = 16
NW = NC * NS
NPW = N // NW
L = 16


def _sc_lookup_body(z_hbm, stab_hbm, htab_hbm, s_out, h_out,
                    z_v, s_v, h_v, stab_v, htab_v):
    wid = lax.axis_index("s") * NC + lax.axis_index("c")
    base = wid * NPW
    pltpu.sync_copy(z_hbm.at[pl.ds(base, NPW)], z_v)
    pltpu.sync_copy(stab_hbm, stab_v)
    pltpu.sync_copy(htab_hbm, htab_v)

    UNROLL = 8

    def body(i, carry):
        for u in range(UNROLL):
            off = (i * UNROLL + u) * L
            idx = z_v[pl.ds(off, L)]
            s_v[pl.ds(off, L)] = plsc.load_gather(stab_v, [idx])
            h_v[pl.ds(off, L)] = plsc.load_gather(htab_v, [idx])
        return carry

    lax.fori_loop(0, NPW // (L * UNROLL), body, 0)
    pltpu.sync_copy(s_v, s_out.at[pl.ds(base, NPW)])
    pltpu.sync_copy(h_v, h_out.at[pl.ds(base, NPW)])


def _tc_body(s_ref, h_ref, x_ref, o_ref):
    s = s_ref[...].reshape(1, BC)
    h = h_ref[...].reshape(1, BC)
    o_ref[...] = x_ref[...] * s + h


def kernel(input, z, scale_table, shift_table):
    xt = input.T
    zi = z.astype(jnp.int32)
    stab = jnp.zeros((TAB,), jnp.float32).at[:54].set(scale_table[:, 0])
    htab = jnp.zeros((TAB,), jnp.float32).at[:54].set(shift_table[:, 0])

    sc_lookup = functools.partial(
        pl.kernel,
        mesh=plsc.VectorSubcoreMesh(core_axis_name="c", subcore_axis_name="s"),
        out_type=(
            jax.ShapeDtypeStruct((N,), jnp.float32),
            jax.ShapeDtypeStruct((N,), jnp.float32),
        ),
        scratch_types=[
            pltpu.VMEM((NPW,), jnp.int32),
            pltpu.VMEM((NPW,), jnp.float32),
            pltpu.VMEM((NPW,), jnp.float32),
            pltpu.VMEM((TAB,), jnp.float32),
            pltpu.VMEM((TAB,), jnp.float32),
        ],
        compiler_params=pltpu.CompilerParams(needs_layout_passes=False),
    )(_sc_lookup_body)
    s_row, h_row = sc_lookup(zi, stab, htab)

    row_spec = pl.BlockSpec((BC,), lambda i: (i,))
    x_spec = pl.BlockSpec((D, BC), lambda i: (0, i))
    out_t = pl.pallas_call(
        _tc_body,
        grid=(N // BC,),
        in_specs=[row_spec, row_spec, x_spec],
        out_specs=x_spec,
        out_shape=jax.ShapeDtypeStruct((D, N), jnp.float32),
    )(s_row, h_row, xt)
    return out_t.T


# speedup vs baseline: 1.0098x; 1.0098x over previous
import functools

import jax
import jax.numpy as jnp
from jax import lax
from jax.experimental import pallas as pl
from jax.experimental.pallas import tpu as pltpu
from jax.experimental.pallas import tpu_sc as plsc

N = 524288
D = 64
TAB = 64
BC = 32768

NH = N // 2
CB = NH // BC

NC = 2
NS = 16
NW = NC * NS
NPW = NH // NW
L = 16


def _sc_lookup_body(z_hbm, stab_hbm, htab_hbm, s_out, h_out,
                    z_v, s_v, h_v, stab_v, htab_v):
    wid = lax.axis_index("s") * NC + lax.axis_index("c")
    base = wid * NPW
    pltpu.sync_copy(z_hbm.at[pl.ds(base, NPW)], z_v)
    pltpu.sync_copy(stab_hbm, stab_v)
    pltpu.sync_copy(htab_hbm, htab_v)

    UNROLL = 8

    def body(i, carry):
        for u in range(UNROLL):
            off = (i * UNROLL + u) * L
            idx = z_v[pl.ds(off, L)]
            s_v[pl.ds(off, L)] = plsc.load_gather(stab_v, [idx])
            h_v[pl.ds(off, L)] = plsc.load_gather(htab_v, [idx])
        return carry

    lax.fori_loop(0, NPW // (L * UNROLL), body, 0)
    pltpu.sync_copy(s_v, s_out.at[pl.ds(base, NPW)])
    pltpu.sync_copy(h_v, h_out.at[pl.ds(base, NPW)])


def _tc_onehot_body(z_ref, stab_ref, htab_ref, x_ref, o_ref):
    zb = z_ref[...].reshape(1, BC)
    k = lax.broadcasted_iota(jnp.int32, (D, BC), 0)
    e = zb == k
    s = jnp.sum(jnp.where(e, stab_ref[...], 0.0), axis=0, keepdims=True)
    h = jnp.sum(jnp.where(e, htab_ref[...], 0.0), axis=0, keepdims=True)
    o_ref[...] = x_ref[...] * s + h


def _tc_rows_body(s_ref, h_ref, x_ref, buf_ref, o_ref):
    del buf_ref
    s = s_ref[...].reshape(1, BC)
    h = h_ref[...].reshape(1, BC)
    o_ref[...] = x_ref[...] * s + h


def kernel(input, z, scale_table, shift_table):
    xt = input.T
    zi = z.astype(jnp.int32)
    stab = jnp.zeros((TAB,), jnp.float32).at[:54].set(scale_table[:, 0])
    htab = jnp.zeros((TAB,), jnp.float32).at[:54].set(shift_table[:, 0])
    stab_col = jnp.zeros((D, 1), jnp.float32).at[:54, 0].set(scale_table[:, 0])
    htab_col = jnp.zeros((D, 1), jnp.float32).at[:54, 0].set(shift_table[:, 0])

    sc_lookup = functools.partial(
        pl.kernel,
        mesh=plsc.VectorSubcoreMesh(core_axis_name="c", subcore_axis_name="s"),
        out_type=(
            jax.ShapeDtypeStruct((NH,), jnp.float32),
            jax.ShapeDtypeStruct((NH,), jnp.float32),
        ),
        scratch_types=[
            pltpu.VMEM((NPW,), jnp.int32),
            pltpu.VMEM((NPW,), jnp.float32),
            pltpu.VMEM((NPW,), jnp.float32),
            pltpu.VMEM((TAB,), jnp.float32),
            pltpu.VMEM((TAB,), jnp.float32),
        ],
        compiler_params=pltpu.CompilerParams(needs_layout_passes=False),
    )(_sc_lookup_body)
    s2, h2 = sc_lookup(zi[NH:], stab, htab)

    row_spec = pl.BlockSpec((BC,), lambda i: (i,))
    tab_spec = pl.BlockSpec((D, 1), lambda i: (0, 0))
    x_lo_spec = pl.BlockSpec((D, BC), lambda i: (0, i))
    x_hi_spec = pl.BlockSpec((D, BC), lambda i: (0, CB + i))

    buf = pl.pallas_call(
        _tc_onehot_body,
        grid=(CB,),
        in_specs=[row_spec, tab_spec, tab_spec, x_lo_spec],
        out_specs=x_lo_spec,
        out_shape=jax.ShapeDtypeStruct((D, N), jnp.float32),
    )(zi[:NH], stab_col, htab_col, xt)

    out_t = pl.pallas_call(
        _tc_rows_body,
        grid=(CB,),
        in_specs=[
            row_spec,
            row_spec,
            x_hi_spec,
            pl.BlockSpec(memory_space=pltpu.HBM),
        ],
        out_specs=x_hi_spec,
        out_shape=jax.ShapeDtypeStruct((D, N), jnp.float32),
        input_output_aliases={3: 0},
    )(s2, h2, xt, buf)
    return out_t.T
